# baseline (speedup 1.0000x reference)
import jax
import jax.numpy as jnp
from jax.experimental import pallas as pl
from jax.experimental.pallas import tpu as pltpu

_P = 16800
_ROWS = 132
_PPAD = _ROWS * 128
_NOBJ = 16
_TH = 0.35
_NEGPOS = 7
_V0 = 0.1
_V1 = 0.2
_B = 32


def _match_kernel(t_ref, pr_ref, loc_ref, conf_ref, lm_ref,
                  out_ref, ce_ref):
    b = pl.program_id(0)
    f32 = jnp.float32

    tv = [[t_ref[0, i, c] for c in range(19)] for i in range(_NOBJ)]

    pcx = pr_ref[0]
    pcy = pr_ref[1]
    pw = pr_ref[2]
    ph = pr_ref[3]
    ipwx = pr_ref[4]
    ipwy = pr_ref[5]
    ppx = pr_ref[6]
    ppy = pr_ref[7]
    lpw = pr_ref[8]
    lph = pr_ref[9]
    px0 = pcx - pw / 2.0
    py0 = pcy - ph / 2.0
    px1 = pcx + pw / 2.0
    py1 = pcy + ph / 2.0
    area_p = (px1 - px0) * (py1 - py0)

    ri = jax.lax.broadcasted_iota(jnp.int32, (_ROWS, 128), 0)
    ci = jax.lax.broadcasted_iota(jnp.int32, (_ROWS, 128), 1)
    fi = ri * 128 + ci
    valid = fi < _P

    ovs = []
    for i in range(_NOBJ):
        tx0, ty0, tx1, ty1, area_t = tv[i][0:5]
        iw = jnp.maximum(jnp.minimum(px1, tx1) - jnp.maximum(px0, tx0), 0.0)
        ih = jnp.maximum(jnp.minimum(py1, ty1) - jnp.maximum(py0, ty0), 0.0)
        inter = iw * ih
        ovs.append(inter / (area_t + area_p - inter))

    pairs = [(ovs[i], jnp.full((_ROWS, 128), i, jnp.int32))
             for i in range(_NOBJ)]
    while len(pairs) > 1:
        nxt = []
        for j in range(0, len(pairs), 2):
            (va, ia), (vb, ib) = pairs[j], pairs[j + 1]
            upd = vb > va
            nxt.append((jnp.where(upd, vb, va), jnp.where(upd, ib, ia)))
        pairs = nxt
    bto, bti = pairs[0]

    ms = [jnp.max(ovs[i]) for i in range(_NOBJ)]
    bpis = [jnp.min(jnp.where(ovs[i] == ms[i], fi, _PPAD))
            for i in range(_NOBJ)]
    for i in range(_NOBJ):
        eq = fi == bpis[i]
        bto = jnp.where(eq, 2.0, bto)
        bti = jnp.where(eq, i, bti)

    pos = bto >= _TH
    npos = jnp.sum(pos.astype(f32))

    zero = jnp.zeros((_ROWS, 128), f32)
    masks = [(bti == i).astype(f32) for i in range(_NOBJ)]

    def gather(c):
        acc = zero
        for i in range(_NOBJ):
            acc = acc + masks[i] * tv[i][c]
        return acc

    g = [
        gather(5) * ipwx - ppx,
        gather(6) * ipwy - ppy,
        gather(7) - lpw,
        gather(8) - lph,
    ]
    ll = zero
    for c in range(4):
        d = jnp.abs(loc_ref[0, c] - g[c])
        ll = ll + jnp.where(d < 1.0, 0.5 * d * d, d - 0.5)
    loss_l = jnp.sum(jnp.where(pos, ll, 0.0))

    llm = zero
    for k in range(5):
        gx = gather(9 + 2 * k) * ipwx - ppx
        gy = gather(10 + 2 * k) * ipwy - ppy
        dx = jnp.abs(lm_ref[0, 2 * k] - gx)
        dy = jnp.abs(lm_ref[0, 2 * k + 1] - gy)
        llm = llm + jnp.where(dx < 1.0, 0.5 * dx * dx, dx - 0.5)
        llm = llm + jnp.where(dy < 1.0, 0.5 * dy * dy, dy - 0.5)
    loss_lm = jnp.sum(jnp.where(pos, llm, 0.0))

    c0 = conf_ref[0, 0]
    c1 = conf_ref[0, 1]
    mx = jnp.maximum(c0, c1)
    lse = mx + jnp.log(jnp.exp(c0 - mx) + jnp.exp(c1 - mx))
    gath = jnp.where(pos, c1, c0)
    ce = jnp.where(valid, lse - gath, 0.0)
    ceb = jax.lax.bitcast_convert_type(ce, jnp.int32)
    ce_ref[0] = jnp.where(pos, ceb | jnp.int32(-2147483648), ceb)

    lane = jax.lax.broadcasted_iota(jnp.int32, (1, 128), 1)
    vec = (jnp.where(lane == 0, loss_l, 0.0)
           + jnp.where(lane == 2, loss_lm, 0.0)
           + jnp.where(lane == 3, npos, 0.0)
           + jnp.where(lane == 8 + b, npos, 0.0))

    @pl.when(b == 0)
    def _():
        out_ref[...] = jnp.zeros_like(out_ref)

    out_ref[...] += vec


def _mine_kernel(np_ref, ce_ref, out_ref):
    f32 = jnp.float32
    sign = jnp.int32(-2147483648)
    rows = []
    poss = []
    ks = []
    his = []
    for b in range(_B):
        packed = ce_ref[b]
        ceb = packed & jnp.int32(2147483647)
        rows.append(ceb)
        poss.append((packed & sign) != 0)
        npos = np_ref[0, 8 + b].astype(jnp.int32)
        ks.append(jnp.minimum(_NEGPOS * npos, _P - 1))
        his.append(jnp.max(ceb))

    los = [jnp.int32(0)] * _B
    for _ in range(31):
        for b in range(_B):
            lo, hi = los[b], his[b]
            mid = lo + (hi - lo + 1) // 2
            cnt = jnp.sum(jnp.where(rows[b] >= mid, 1, 0))
            big = cnt >= ks[b]
            los[b] = jnp.where(big, mid, lo)
            his[b] = jnp.where(big, hi, mid - 1)

    loss_c = jnp.float32(0.0)
    for b in range(_B):
        sel = jnp.logical_or(poss[b], rows[b] >= los[b])
        ce = jax.lax.bitcast_convert_type(rows[b], f32)
        loss_c = loss_c + jnp.sum(jnp.where(sel, ce, 0.0))

    lane = jax.lax.broadcasted_iota(jnp.int32, (1, 128), 1)
    out_ref[...] = jnp.where(lane == 1, loss_c, 0.0)


def _planar(x):
    b, p, c = x.shape
    xp = jnp.pad(x, ((0, 0), (0, _PPAD - p), (0, 0)))
    xt = jnp.stack([xp[:, :, i] for i in range(c)], axis=1)
    return xt.reshape(b, c, _ROWS, 128)


def kernel(loc_data, conf_data, landm_data, priors, targets):
    num = loc_data.shape[0]
    f32 = jnp.float32
    locp = _planar(loc_data)
    confp = _planar(conf_data)
    lmp = _planar(landm_data)

    pad = jnp.tile(jnp.array([[2.0, 2.0, 0.0, 0.0]], f32),
                   (_PPAD - _P, 1))
    prpad = jnp.concatenate([priors, pad], axis=0)
    pcx, pcy, pw, ph = (prpad[:, i] for i in range(4))
    pr_ext = jnp.stack([
        pcx, pcy, pw, ph,
        1.0 / (_V0 * pw), 1.0 / (_V0 * ph),
        pcx / (_V0 * pw), pcy / (_V0 * ph),
        jnp.log(pw) / _V1, jnp.log(ph) / _V1,
    ], axis=0).reshape(10, _ROWS, 128)

    tx0, ty0, tx1, ty1 = (targets[:, :, i] for i in range(4))
    t_ext = jnp.concatenate([
        targets[:, :, 0:4],
        ((tx1 - tx0) * (ty1 - ty0))[..., None],
        ((tx0 + tx1) / 2.0)[..., None],
        ((ty0 + ty1) / 2.0)[..., None],
        (jnp.log(tx1 - tx0) / _V1)[..., None],
        (jnp.log(ty1 - ty0) / _V1)[..., None],
        targets[:, :, 4:14],
    ], axis=2)

    outa, ceb = pl.pallas_call(
        _match_kernel,
        grid=(num,),
        in_specs=[
            pl.BlockSpec((1, _NOBJ, 19), lambda b: (b, 0, 0)),
            pl.BlockSpec((10, _ROWS, 128), lambda b: (0, 0, 0)),
            pl.BlockSpec((1, 4, _ROWS, 128), lambda b: (b, 0, 0, 0)),
            pl.BlockSpec((1, 2, _ROWS, 128), lambda b: (b, 0, 0, 0)),
            pl.BlockSpec((1, 10, _ROWS, 128), lambda b: (b, 0, 0, 0)),
        ],
        out_specs=[
            pl.BlockSpec((1, 128), lambda b: (0, 0)),
            pl.BlockSpec((1, _ROWS, 128), lambda b: (b, 0, 0)),
        ],
        out_shape=[
            jax.ShapeDtypeStruct((1, 128), f32),
            jax.ShapeDtypeStruct((num, _ROWS, 128), jnp.int32),
        ],
        compiler_params=pltpu.CompilerParams(
            dimension_semantics=("arbitrary",)),
    )(t_ext, pr_ext, locp, confp, lmp)

    outb = pl.pallas_call(
        _mine_kernel,
        grid=(1,),
        in_specs=[
            pl.BlockSpec((1, 128), lambda b: (0, 0)),
            pl.BlockSpec((num, _ROWS, 128), lambda b: (0, 0, 0)),
        ],
        out_specs=pl.BlockSpec((1, 128), lambda b: (0, 0)),
        out_shape=jax.ShapeDtypeStruct((1, 128), f32),
        compiler_params=pltpu.CompilerParams(
            dimension_semantics=("arbitrary",)),
    )(outa, ceb)

    sa = outa[0]
    n = jnp.maximum(sa[3], 1.0)
    return (sa[0] / n, outb[0, 1] / n, sa[2] / n)


# speedup vs baseline: 1.1090x; 1.1090x over previous
import jax
import jax.numpy as jnp
from jax.experimental import pallas as pl
from jax.experimental.pallas import tpu as pltpu

_P = 16800
_ROWS = 132
_PPAD = _ROWS * 128
_NOBJ = 16
_TH = 0.35
_NEGPOS = 7
_V0 = 0.1
_V1 = 0.2
_B = 32


def _match_kernel(t_ref, pr_ref, loc_ref, conf_ref, lm_ref,
                  out_ref, ce_ref):
    b = pl.program_id(0)
    f32 = jnp.float32

    tv = [[t_ref[0, i, c] for c in range(19)] for i in range(_NOBJ)]

    pcx = pr_ref[0]
    pcy = pr_ref[1]
    pw = pr_ref[2]
    ph = pr_ref[3]
    ipwx = pr_ref[4]
    ipwy = pr_ref[5]
    ppx = pr_ref[6]
    ppy = pr_ref[7]
    lpw = pr_ref[8]
    lph = pr_ref[9]
    px0 = pcx - pw / 2.0
    py0 = pcy - ph / 2.0
    px1 = pcx + pw / 2.0
    py1 = pcy + ph / 2.0
    area_p = (px1 - px0) * (py1 - py0)

    ri = jax.lax.broadcasted_iota(jnp.int32, (_ROWS, 128), 0)
    ci = jax.lax.broadcasted_iota(jnp.int32, (_ROWS, 128), 1)
    fi = ri * 128 + ci
    valid = fi < _P

    ovs = []
    for i in range(_NOBJ):
        tx0, ty0, tx1, ty1, area_t = tv[i][0:5]
        iw = jnp.maximum(jnp.minimum(px1, tx1) - jnp.maximum(px0, tx0), 0.0)
        ih = jnp.maximum(jnp.minimum(py1, ty1) - jnp.maximum(py0, ty0), 0.0)
        inter = iw * ih
        ovs.append(inter / (area_t + area_p - inter))

    pairs = [(ovs[i], jnp.full((_ROWS, 128), i, jnp.int32))
             for i in range(_NOBJ)]
    while len(pairs) > 1:
        nxt = []
        for j in range(0, len(pairs), 2):
            (va, ia), (vb, ib) = pairs[j], pairs[j + 1]
            upd = vb > va
            nxt.append((jnp.where(upd, vb, va), jnp.where(upd, ib, ia)))
        pairs = nxt
    bto, bti = pairs[0]

    ms = [jnp.max(ovs[i]) for i in range(_NOBJ)]
    bpis = [jnp.min(jnp.where(ovs[i] == ms[i], fi, _PPAD))
            for i in range(_NOBJ)]
    for i in range(_NOBJ):
        eq = fi == bpis[i]
        bto = jnp.where(eq, 2.0, bto)
        bti = jnp.where(eq, i, bti)

    pos = bto >= _TH
    npos = jnp.sum(pos.astype(f32))

    zero = jnp.zeros((_ROWS, 128), f32)
    masks = [(bti == i).astype(f32) for i in range(_NOBJ)]

    def gather(c):
        acc = zero
        for i in range(_NOBJ):
            acc = acc + masks[i] * tv[i][c]
        return acc

    g = [
        gather(5) * ipwx - ppx,
        gather(6) * ipwy - ppy,
        gather(7) - lpw,
        gather(8) - lph,
    ]
    ll = zero
    for c in range(4):
        d = jnp.abs(loc_ref[0, c] - g[c])
        ll = ll + jnp.where(d < 1.0, 0.5 * d * d, d - 0.5)
    loss_l = jnp.sum(jnp.where(pos, ll, 0.0))

    llm = zero
    for k in range(5):
        gx = gather(9 + 2 * k) * ipwx - ppx
        gy = gather(10 + 2 * k) * ipwy - ppy
        dx = jnp.abs(lm_ref[0, 2 * k] - gx)
        dy = jnp.abs(lm_ref[0, 2 * k + 1] - gy)
        llm = llm + jnp.where(dx < 1.0, 0.5 * dx * dx, dx - 0.5)
        llm = llm + jnp.where(dy < 1.0, 0.5 * dy * dy, dy - 0.5)
    loss_lm = jnp.sum(jnp.where(pos, llm, 0.0))

    c0 = conf_ref[0, 0]
    c1 = conf_ref[0, 1]
    mx = jnp.maximum(c0, c1)
    lse = mx + jnp.log(jnp.exp(c0 - mx) + jnp.exp(c1 - mx))
    gath = jnp.where(pos, c1, c0)
    ce = jnp.where(valid, lse - gath, 0.0)
    ceb = jax.lax.bitcast_convert_type(ce, jnp.int32)
    ce_ref[0] = jnp.where(pos, ceb | jnp.int32(-2147483648), ceb)

    lane = jax.lax.broadcasted_iota(jnp.int32, (1, 128), 1)
    vec = (jnp.where(lane == 0, loss_l, 0.0)
           + jnp.where(lane == 2, loss_lm, 0.0)
           + jnp.where(lane == 3, npos, 0.0)
           + jnp.where(lane == 8 + b, npos, 0.0))

    @pl.when(b == 0)
    def _():
        out_ref[...] = jnp.zeros_like(out_ref)

    out_ref[...] += vec


def _mine_kernel(np_ref, ce_ref, out_ref):
    f32 = jnp.float32
    sign = jnp.int32(-2147483648)
    rows = []
    poss = []
    ks = []
    his = []
    for b in range(_B):
        packed = ce_ref[b]
        ceb = packed & jnp.int32(2147483647)
        rows.append(ceb)
        poss.append((packed & sign) != 0)
        npos = np_ref[0, 8 + b].astype(jnp.int32)
        ks.append(jnp.minimum(_NEGPOS * npos, _P - 1))
        his.append(jnp.max(ceb))

    los = [jnp.int32(0)] * _B
    for _ in range(31):
        for b in range(_B):
            lo, hi = los[b], his[b]
            mid = lo + (hi - lo + 1) // 2
            cnt = jnp.sum(jnp.where(rows[b] >= mid, 1, 0))
            big = cnt >= ks[b]
            los[b] = jnp.where(big, mid, lo)
            his[b] = jnp.where(big, hi, mid - 1)

    loss_c = jnp.float32(0.0)
    for b in range(_B):
        sel = jnp.logical_or(poss[b], rows[b] >= los[b])
        ce = jax.lax.bitcast_convert_type(rows[b], f32)
        loss_c = loss_c + jnp.sum(jnp.where(sel, ce, 0.0))

    lane = jax.lax.broadcasted_iota(jnp.int32, (1, 128), 1)
    out_ref[...] = jnp.where(lane == 1, loss_c, 0.0)


def _planar(x):
    b, p, c = x.shape
    xp = jnp.pad(x, ((0, 0), (0, _PPAD - p), (0, 0)))
    xt = jnp.moveaxis(xp, 2, 1)
    return xt.reshape(b, c, _ROWS, 128)


def kernel(loc_data, conf_data, landm_data, priors, targets):
    num = loc_data.shape[0]
    f32 = jnp.float32
    locp = _planar(loc_data)
    confp = _planar(conf_data)
    lmp = _planar(landm_data)

    pad = jnp.tile(jnp.array([[2.0, 2.0, 0.0, 0.0]], f32),
                   (_PPAD - _P, 1))
    prpad = jnp.concatenate([priors, pad], axis=0)
    pcx, pcy, pw, ph = (prpad[:, i] for i in range(4))
    pr_ext = jnp.stack([
        pcx, pcy, pw, ph,
        1.0 / (_V0 * pw), 1.0 / (_V0 * ph),
        pcx / (_V0 * pw), pcy / (_V0 * ph),
        jnp.log(pw) / _V1, jnp.log(ph) / _V1,
    ], axis=0).reshape(10, _ROWS, 128)

    tx0, ty0, tx1, ty1 = (targets[:, :, i] for i in range(4))
    t_ext = jnp.concatenate([
        targets[:, :, 0:4],
        ((tx1 - tx0) * (ty1 - ty0))[..., None],
        ((tx0 + tx1) / 2.0)[..., None],
        ((ty0 + ty1) / 2.0)[..., None],
        (jnp.log(tx1 - tx0) / _V1)[..., None],
        (jnp.log(ty1 - ty0) / _V1)[..., None],
        targets[:, :, 4:14],
    ], axis=2)

    outa, ceb = pl.pallas_call(
        _match_kernel,
        grid=(num,),
        in_specs=[
            pl.BlockSpec((1, _NOBJ, 19), lambda b: (b, 0, 0)),
            pl.BlockSpec((10, _ROWS, 128), lambda b: (0, 0, 0)),
            pl.BlockSpec((1, 4, _ROWS, 128), lambda b: (b, 0, 0, 0)),
            pl.BlockSpec((1, 2, _ROWS, 128), lambda b: (b, 0, 0, 0)),
            pl.BlockSpec((1, 10, _ROWS, 128), lambda b: (b, 0, 0, 0)),
        ],
        out_specs=[
            pl.BlockSpec((1, 128), lambda b: (0, 0)),
            pl.BlockSpec((1, _ROWS, 128), lambda b: (b, 0, 0)),
        ],
        out_shape=[
            jax.ShapeDtypeStruct((1, 128), f32),
            jax.ShapeDtypeStruct((num, _ROWS, 128), jnp.int32),
        ],
        compiler_params=pltpu.CompilerParams(
            dimension_semantics=("arbitrary",)),
    )(t_ext, pr_ext, locp, confp, lmp)

    outb = pl.pallas_call(
        _mine_kernel,
        grid=(1,),
        in_specs=[
            pl.BlockSpec((1, 128), lambda b: (0, 0)),
            pl.BlockSpec((num, _ROWS, 128), lambda b: (0, 0, 0)),
        ],
        out_specs=pl.BlockSpec((1, 128), lambda b: (0, 0)),
        out_shape=jax.ShapeDtypeStruct((1, 128), f32),
        compiler_params=pltpu.CompilerParams(
            dimension_semantics=("arbitrary",)),
    )(outa, ceb)

    sa = outa[0]
    n = jnp.maximum(sa[3], 1.0)
    return (sa[0] / n, outb[0, 1] / n, sa[2] / n)
